# baseline (speedup 1.0000x reference)
import jax
import jax.numpy as jnp
from jax.experimental import pallas as pl
from jax.experimental.pallas import tpu as pltpu

_NUM_INTERVENTIONS = 128
_ROWS = 2048
_NB = 8
_NBUF = 3


def _body(mask_ref, x_hbm, c_hbm, o_hbm, xb, cb, ob, in_sems, out_sems):
    m = mask_ref[...]

    def start_in(k):
        s = k % _NBUF
        pltpu.make_async_copy(
            x_hbm.at[pl.ds(k * _ROWS, _ROWS)], xb.at[s], in_sems.at[0, s]
        ).start()
        pltpu.make_async_copy(
            c_hbm.at[pl.ds(k * _ROWS, _ROWS)], cb.at[s], in_sems.at[1, s]
        ).start()

    for k in range(_NBUF):
        start_in(k)

    for k in range(_NB):
        s = k % _NBUF
        pltpu.make_async_copy(
            x_hbm.at[pl.ds(k * _ROWS, _ROWS)], xb.at[s], in_sems.at[0, s]
        ).wait()
        pltpu.make_async_copy(
            c_hbm.at[pl.ds(k * _ROWS, _ROWS)], cb.at[s], in_sems.at[1, s]
        ).wait()
        if k >= _NBUF:
            pltpu.make_async_copy(
                ob.at[s], o_hbm.at[pl.ds((k - _NBUF) * _ROWS, _ROWS)], out_sems.at[s]
            ).wait()
        ob[s] = jnp.where(m > 0.5, 1.0 - cb[s], xb[s])
        pltpu.make_async_copy(
            ob.at[s], o_hbm.at[pl.ds(k * _ROWS, _ROWS)], out_sems.at[s]
        ).start()
        if k + _NBUF < _NB:
            start_in(k + _NBUF)

    for k in range(_NB - _NBUF, _NB):
        s = k % _NBUF
        pltpu.make_async_copy(
            ob.at[s], o_hbm.at[pl.ds(k * _ROWS, _ROWS)], out_sems.at[s]
        ).wait()


def kernel(x, concepts):
    batch, dim = x.shape
    idx = jax.random.permutation(jax.random.key(42), dim)[:_NUM_INTERVENTIONS]
    mask = jnp.zeros((1, dim), jnp.float32).at[0, idx].set(1.0)

    return pl.pallas_call(
        _body,
        in_specs=[
            pl.BlockSpec(memory_space=pltpu.MemorySpace.VMEM),
            pl.BlockSpec(memory_space=pl.ANY),
            pl.BlockSpec(memory_space=pl.ANY),
        ],
        out_specs=pl.BlockSpec(memory_space=pl.ANY),
        out_shape=jax.ShapeDtypeStruct((batch, dim), x.dtype),
        scratch_shapes=[
            pltpu.VMEM((_NBUF, _ROWS, dim), jnp.float32),
            pltpu.VMEM((_NBUF, _ROWS, dim), jnp.float32),
            pltpu.VMEM((_NBUF, _ROWS, dim), jnp.float32),
            pltpu.SemaphoreType.DMA((2, _NBUF)),
            pltpu.SemaphoreType.DMA((_NBUF,)),
        ],
    )(mask, x, concepts)


# speedup vs baseline: 1.0090x; 1.0090x over previous
import jax
import jax.numpy as jnp
from jax.experimental import pallas as pl
from jax.experimental.pallas import tpu as pltpu

_NUM_INTERVENTIONS = 128
_ROWS = 1024
_NB = 16
_NBUF = 4


def _body(mask_ref, x_hbm, c_hbm, o_hbm, xb, cb, ob, in_sems, out_sems):
    m = mask_ref[...]

    def start_in(k):
        s = k % _NBUF
        pltpu.make_async_copy(
            x_hbm.at[pl.ds(k * _ROWS, _ROWS)], xb.at[s], in_sems.at[0, s]
        ).start()
        pltpu.make_async_copy(
            c_hbm.at[pl.ds(k * _ROWS, _ROWS)], cb.at[s], in_sems.at[1, s]
        ).start()

    for k in range(_NBUF):
        start_in(k)

    for k in range(_NB):
        s = k % _NBUF
        pltpu.make_async_copy(
            x_hbm.at[pl.ds(k * _ROWS, _ROWS)], xb.at[s], in_sems.at[0, s]
        ).wait()
        pltpu.make_async_copy(
            c_hbm.at[pl.ds(k * _ROWS, _ROWS)], cb.at[s], in_sems.at[1, s]
        ).wait()
        if k >= _NBUF:
            pltpu.make_async_copy(
                ob.at[s], o_hbm.at[pl.ds((k - _NBUF) * _ROWS, _ROWS)], out_sems.at[s]
            ).wait()
        ob[s] = jnp.where(m > 0.5, 1.0 - cb[s], xb[s])
        pltpu.make_async_copy(
            ob.at[s], o_hbm.at[pl.ds(k * _ROWS, _ROWS)], out_sems.at[s]
        ).start()
        if k + _NBUF < _NB:
            start_in(k + _NBUF)

    for k in range(_NB - _NBUF, _NB):
        s = k % _NBUF
        pltpu.make_async_copy(
            ob.at[s], o_hbm.at[pl.ds(k * _ROWS, _ROWS)], out_sems.at[s]
        ).wait()


def kernel(x, concepts):
    batch, dim = x.shape
    idx = jax.random.permutation(jax.random.key(42), dim)[:_NUM_INTERVENTIONS]
    mask = jnp.zeros((1, dim), jnp.float32).at[0, idx].set(1.0)

    return pl.pallas_call(
        _body,
        in_specs=[
            pl.BlockSpec(memory_space=pltpu.MemorySpace.VMEM),
            pl.BlockSpec(memory_space=pl.ANY),
            pl.BlockSpec(memory_space=pl.ANY),
        ],
        out_specs=pl.BlockSpec(memory_space=pl.ANY),
        out_shape=jax.ShapeDtypeStruct((batch, dim), x.dtype),
        scratch_shapes=[
            pltpu.VMEM((_NBUF, _ROWS, dim), jnp.float32),
            pltpu.VMEM((_NBUF, _ROWS, dim), jnp.float32),
            pltpu.VMEM((_NBUF, _ROWS, dim), jnp.float32),
            pltpu.SemaphoreType.DMA((2, _NBUF)),
            pltpu.SemaphoreType.DMA((_NBUF,)),
        ],
    )(mask, x, concepts)
